# baseline (speedup 1.0000x reference)
import jax
import jax.numpy as jnp
from jax.experimental import pallas as pl
from jax.experimental.pallas import tpu as pltpu


def _body(ms_ref, ml_ref, mean_ref, std_ref, mo_ref, so_ref, *, BT, D):
    b = pl.program_id(0)
    tb = pl.program_id(1)
    start = ms_ref[b]
    mlen = ml_ref[0]
    t0 = tb * BT
    tids = t0 + jax.lax.broadcasted_iota(jnp.int32, (1, BT, D), 1)
    m = (tids >= start) & (tids < start + mlen)
    mo_ref[...] = jnp.where(m, jnp.float32(0.0), mean_ref[...])
    so_ref[...] = jnp.where(m, jnp.float32(1.0), std_ref[...])


def kernel(mean, std, mask_start, mask_len):
    B, T, D = mean.shape
    BT = 1024
    ms = jnp.asarray(mask_start, jnp.int32)
    ml = jnp.asarray(mask_len, jnp.int32).reshape((1,))

    import functools
    body = functools.partial(_body, BT=BT, D=D)
    data_spec = pl.BlockSpec((1, BT, D), lambda b, tb: (b, tb, 0))
    out = pl.pallas_call(
        body,
        grid=(B, T // BT),
        in_specs=[
            pl.BlockSpec(memory_space=pltpu.SMEM),
            pl.BlockSpec(memory_space=pltpu.SMEM),
            data_spec,
            data_spec,
        ],
        out_specs=[data_spec, data_spec],
        out_shape=[
            jax.ShapeDtypeStruct((B, T, D), jnp.float32),
            jax.ShapeDtypeStruct((B, T, D), jnp.float32),
        ],
    )(ms, ml, mean, std)
    return (out[0], out[1])
